# Initial kernel scaffold; baseline (speedup 1.0000x reference)
import jax
import jax.numpy as jnp
from jax.experimental import pallas as pl


def kernel(features, labels):
    raise NotImplementedError("write your pallas kernel here")


# speedup vs baseline: 4.8144x; 4.8144x over previous
import functools

import jax
import jax.numpy as jnp
from jax import lax
from jax.experimental import pallas as pl
from jax.experimental.pallas import tpu as pltpu
from jax.experimental.pallas import tpu_sc as plsc

B = 16384
D = 128
C = 1000
NC = 2
NS = 16
NW = NC * NS
ROWS_PER_TILE = B // NW
CHUNK = 128
NCHUNK = ROWS_PER_TILE // CHUNK
CP = 1024
CROWS = CP // NS
CNTW = 16


def _sc_segment_sum(features, labels2d):
    mesh = plsc.VectorSubcoreMesh(core_axis_name="c", subcore_axis_name="s")

    @functools.partial(
        pl.kernel,
        mesh=mesh,
        compiler_params=pltpu.CompilerParams(needs_layout_passes=False),
        out_type=(
            jax.ShapeDtypeStruct((NC, CP, D), jnp.float32),
            jax.ShapeDtypeStruct((NW, CP), jnp.float32),
        ),
        scratch_types=[
            pltpu.VMEM((ROWS_PER_TILE, D), jnp.float32),
            pltpu.VMEM((NCHUNK, CHUNK), jnp.int32),
            pltpu.VMEM((CP,), jnp.float32),
            pltpu.VMEM((CROWS, D), jnp.float32),
            pltpu.VMEM_SHARED((CP, D), jnp.float32),
        ],
    )
    def k(feat_hbm, lbl_hbm, sum_out, cnt_out,
          feat_v, lbl_v, cnt_v, zs_v, acc_s):
        cid = lax.axis_index("c")
        sid = lax.axis_index("s")
        wid = cid * NS + sid
        base = wid * ROWS_PER_TILE

        zeros16 = jnp.zeros((16,), jnp.float32)
        ones16 = jnp.ones((16,), jnp.float32)

        def fill_zs(i, _):
            for j in range(D // 16):
                zs_v[i, pl.ds(j * 16, 16)] = zeros16
            return 0

        lax.fori_loop(0, CROWS, fill_zs, 0)

        def fill_zc(i, _):
            cnt_v[pl.ds(i * 16, 16)] = zeros16
            return 0

        lax.fori_loop(0, CP // 16, fill_zc, 0)

        pltpu.sync_copy(zs_v, acc_s.at[pl.ds(sid * CROWS, CROWS)])

        pltpu.sync_copy(feat_hbm.at[pl.ds(base, ROWS_PER_TILE)], feat_v)
        pltpu.sync_copy(lbl_hbm.at[pl.ds(wid * NCHUNK, NCHUNK)], lbl_v)

        plsc.subcore_barrier()

        for j in range(NCHUNK):
            idx = lbl_v.at[j]
            pltpu.sync_copy(feat_v.at[pl.ds(j * CHUNK, CHUNK)],
                            acc_s.at[idx], add=True)

        def count_step(v, _):
            lbl16 = lbl_v[v // 8, pl.ds((v % 8) * 16, 16)]
            plsc.addupdate_scatter(cnt_v, [lbl16], ones16)
            return 0

        lax.fori_loop(0, ROWS_PER_TILE // 16, count_step, 0)

        plsc.subcore_barrier()

        r0 = sid * CROWS
        pltpu.sync_copy(acc_s.at[pl.ds(r0, CROWS)],
                        sum_out.at[cid, pl.ds(r0, CROWS)])
        pltpu.sync_copy(cnt_v, cnt_out.at[wid])

    return k(features, labels2d)


def _combine(psum, pcnt):
    def body(ps_ref, pc_ref, o_ref):
        s = ps_ref[0] + ps_ref[1]
        c = jnp.sum(pc_ref[...], axis=0, keepdims=True)
        ct = jnp.transpose(c, (1, 0))
        s = s[:C]
        ct = ct[:C]
        o_ref[...] = jnp.where(ct > 0, s / jnp.maximum(ct, 1.0),
                               jnp.zeros_like(s))

    return pl.pallas_call(
        body,
        out_shape=jax.ShapeDtypeStruct((C, D), jnp.float32),
    )(psum, pcnt)


def kernel(features, labels):
    labels2d = labels.reshape(B // CHUNK, CHUNK)
    psum, pcnt = _sc_segment_sum(features, labels2d)
    return _combine(psum, pcnt)


# speedup vs baseline: 5.1261x; 1.0647x over previous
import functools

import jax
import jax.numpy as jnp
from jax import lax
from jax.experimental import pallas as pl
from jax.experimental.pallas import tpu as pltpu
from jax.experimental.pallas import tpu_sc as plsc

B = 16384
D = 128
C = 1000
NC = 2
NS = 16
NW = NC * NS
ROWS_PER_TILE = B // NW
CHUNK = 128
NCHUNK = ROWS_PER_TILE // CHUNK
CP = 1024
CROWS = CP // NS
CNTW = 16


def _sc_segment_sum(features, labels2d):
    mesh = plsc.VectorSubcoreMesh(core_axis_name="c", subcore_axis_name="s")

    @functools.partial(
        pl.kernel,
        mesh=mesh,
        compiler_params=pltpu.CompilerParams(needs_layout_passes=False),
        out_type=(
            jax.ShapeDtypeStruct((NC, CP, D), jnp.float32),
            jax.ShapeDtypeStruct((NW, CP), jnp.float32),
        ),
        scratch_types=[
            pltpu.VMEM((ROWS_PER_TILE, D), jnp.float32),
            pltpu.VMEM((NCHUNK, CHUNK), jnp.int32),
            pltpu.VMEM((CP,), jnp.float32),
            pltpu.VMEM((CROWS, D), jnp.float32),
            pltpu.VMEM_SHARED((CP, D), jnp.float32),
            pltpu.SemaphoreType.DMA,
            [pltpu.SemaphoreType.DMA] * NCHUNK,
            pltpu.SemaphoreType.DMA,
        ],
    )
    def k(feat_hbm, lbl_hbm, sum_out, cnt_out,
          feat_v, lbl_v, cnt_v, zs_v, acc_s, sem_l, sems_f, sem_s):
        cid = lax.axis_index("c")
        sid = lax.axis_index("s")
        wid = cid * NS + sid
        base = wid * ROWS_PER_TILE

        lcop = pltpu.async_copy(
            lbl_hbm.at[pl.ds(wid * NCHUNK, NCHUNK)], lbl_v, sem_l)
        fcops = [
            pltpu.async_copy(
                feat_hbm.at[pl.ds(base + j * CHUNK, CHUNK)],
                feat_v.at[pl.ds(j * CHUNK, CHUNK)], sems_f[j])
            for j in range(NCHUNK)
        ]

        zeros16 = jnp.zeros((16,), jnp.float32)
        ones16 = jnp.ones((16,), jnp.float32)

        def fill_zs(i, _):
            for j in range(D // 16):
                zs_v[i, pl.ds(j * 16, 16)] = zeros16
            return 0

        lax.fori_loop(0, CROWS, fill_zs, 0)

        def fill_zc(i, _):
            cnt_v[pl.ds(i * 16, 16)] = zeros16
            return 0

        lax.fori_loop(0, CP // 16, fill_zc, 0)

        pltpu.sync_copy(zs_v, acc_s.at[pl.ds(sid * CROWS, CROWS)])

        lcop.wait()

        def count_step(v, _):
            lbl16 = lbl_v[v // 8, pl.ds((v % 8) * 16, 16)]
            plsc.addupdate_scatter(cnt_v, [lbl16], ones16)
            return 0

        lax.fori_loop(0, ROWS_PER_TILE // 16, count_step, 0)

        plsc.subcore_barrier()

        scats = []
        for j in range(NCHUNK):
            fcops[j].wait()
            scats.append(pltpu.async_copy(
                feat_v.at[pl.ds(j * CHUNK, CHUNK)],
                acc_s.at[lbl_v.at[j]], sem_s, add=True))
        for s in scats:
            s.wait()

        plsc.subcore_barrier()

        r0 = sid * CROWS
        pltpu.sync_copy(acc_s.at[pl.ds(r0, CROWS)],
                        sum_out.at[cid, pl.ds(r0, CROWS)])
        pltpu.sync_copy(cnt_v, cnt_out.at[wid])

    return k(features, labels2d)


def _combine(psum, pcnt):
    def body(ps_ref, pc_ref, o_ref):
        s = ps_ref[0] + ps_ref[1]
        c = jnp.sum(pc_ref[...], axis=0, keepdims=True)
        ct = jnp.transpose(c, (1, 0))
        s = s[:C]
        ct = ct[:C]
        o_ref[...] = jnp.where(ct > 0, s / jnp.maximum(ct, 1.0),
                               jnp.zeros_like(s))

    return pl.pallas_call(
        body,
        out_shape=jax.ShapeDtypeStruct((C, D), jnp.float32),
    )(psum, pcnt)


def kernel(features, labels):
    labels2d = labels.reshape(B // CHUNK, CHUNK)
    psum, pcnt = _sc_segment_sum(features, labels2d)
    return _combine(psum, pcnt)
